# Initial kernel scaffold; baseline (speedup 1.0000x reference)
import jax
import jax.numpy as jnp
from jax.experimental import pallas as pl


def kernel(x, edge_index, W, b):
    raise NotImplementedError("write your pallas kernel here")


# speedup vs baseline: 6.0880x; 6.0880x over previous
import jax
import jax.numpy as jnp
from jax import lax
from jax.experimental import pallas as pl
from jax.experimental.pallas import tpu as pltpu
from jax.experimental.pallas import tpu_sc as plsc

NC = 2
NS = 16
CHUNK = 80


def _sc_body(x_hbm, src_hbm, dst_hbm, p_hbm, dg_hbm,
             acc_sh, src_v, dst_v, idx_v, rows_v, deg_l, sem):
    c = lax.axis_index("c")
    s = lax.axis_index("s")

    n_nodes = acc_sh.shape[0]
    d = rows_v.shape[1]
    total_chunks = n_nodes // CHUNK
    cpt = -(-total_chunks // NS)
    last = total_chunks - cpt * (NS - 1)
    nk = jnp.where(s == NS - 1, last, cpt)

    zero16 = jnp.zeros((16,), jnp.float32)
    zero16i = jnp.zeros((16,), jnp.int32)
    one16 = jnp.ones((16,), jnp.float32)
    iota16 = lax.iota(jnp.int32, 16)

    def _fill(i, carry):
        for j in range(d // 16):
            rows_v[i, pl.ds(j * 16, 16)] = zero16
        return carry

    lax.fori_loop(0, CHUNK, _fill, 0)

    def _zdeg(i, carry):
        deg_l[pl.ds(i * 16, 16)] = zero16
        return carry

    lax.fori_loop(0, n_nodes // 16, _zdeg, 0)

    def _set_idx(base):
        for i in range(CHUNK // 16):
            idx_v[pl.ds(i * 16, 16)] = base + (i * 16) + iota16

    def _zero_body(k, carry):
        _set_idx((s * cpt + k) * CHUNK)
        pltpu.sync_copy(rows_v, acc_sh.at[idx_v])
        return carry

    lax.fori_loop(0, nk, _zero_body, 0)

    plsc.subcore_barrier()

    n_edges = src_hbm.shape[0]
    edges_per_tile = n_edges // (NC * NS)
    n_chunks = edges_per_tile // CHUNK
    ebase = (c * NS + s) * edges_per_tile

    def chunk_body(i, carry):
        base = ebase + i * CHUNK
        pltpu.sync_copy(src_hbm.at[pl.ds(base, CHUNK)], src_v)
        pltpu.sync_copy(dst_hbm.at[pl.ds(base, CHUNK)], dst_v)
        pltpu.async_copy(x_hbm.at[src_v], rows_v, sem).wait()
        pltpu.sync_copy(rows_v, acc_sh.at[dst_v], add=True)
        for j in range(CHUNK // 16):
            dst16 = dst_v[pl.ds(j * 16, 16)]
            plsc.addupdate_scatter(deg_l, [dst16], one16)
        return carry

    lax.fori_loop(0, n_chunks, chunk_body, 0)

    plsc.subcore_barrier()

    def _wb_body(k, carry):
        base = (s * cpt + k) * CHUNK
        _set_idx(base)
        pltpu.async_copy(acc_sh.at[idx_v], rows_v, sem).wait()
        hb = pl.multiple_of(base, CHUNK)
        pltpu.sync_copy(rows_v, p_hbm.at[c, pl.ds(hb, CHUNK)])
        return carry

    lax.fori_loop(0, nk, _wb_body, 0)
    pltpu.sync_copy(deg_l, dg_hbm.at[c * NS + s, 0])


def _tc_body(p_ref, dg_ref, w_ref, b_ref, o_ref):
    nw = dg_ref.shape[1]
    sdat = p_ref[0] + p_ref[1]
    ones_col = jnp.ones((nw, 1), jnp.float32)
    deg = lax.dot_general(dg_ref[0], ones_col, (((0,), (0,)), ((), ())),
                          preferred_element_type=jnp.float32)
    m = lax.dot_general(sdat, w_ref[...], (((1,), (1,)), ((), ())),
                        preferred_element_type=jnp.float32)
    o_ref[...] = (m + deg * b_ref[...]) / jnp.maximum(deg, 1.0)


def _sc_aggregate(x, src, dst):
    n_nodes, d = x.shape
    mesh = plsc.VectorSubcoreMesh(core_axis_name="c", subcore_axis_name="s",
                                  num_cores=NC, num_subcores=NS)
    f = pl.kernel(
        _sc_body,
        out_type=[
            jax.ShapeDtypeStruct((NC, n_nodes, d), jnp.float32),
            jax.ShapeDtypeStruct((NC * NS, 1, n_nodes), jnp.float32),
        ],
        mesh=mesh,
        compiler_params=pltpu.CompilerParams(needs_layout_passes=False),
        scratch_types=[
            pltpu.VMEM_SHARED((n_nodes, d), jnp.float32),
            pltpu.VMEM((CHUNK,), jnp.int32),
            pltpu.VMEM((CHUNK,), jnp.int32),
            pltpu.VMEM((CHUNK,), jnp.int32),
            pltpu.VMEM((CHUNK, d), jnp.float32),
            pltpu.VMEM((n_nodes,), jnp.float32),
            pltpu.SemaphoreType.DMA,
        ],
    )
    return f(x, src, dst)


def _tc_finish(p, dg3, w, b2):
    nc, n_nodes, d = p.shape
    nblk, nw, bm = dg3.shape
    grid = (n_nodes // bm,)
    return pl.pallas_call(
        _tc_body,
        grid=grid,
        in_specs=[
            pl.BlockSpec((nc, bm, d), lambda i: (0, i, 0)),
            pl.BlockSpec((1, nw, bm), lambda i: (i, 0, 0)),
            pl.BlockSpec((d, d), lambda i: (0, 0)),
            pl.BlockSpec((1, d), lambda i: (0, 0)),
        ],
        out_specs=pl.BlockSpec((bm, d), lambda i: (i, 0)),
        out_shape=jax.ShapeDtypeStruct((n_nodes, d), jnp.float32),
    )(p, dg3, w, b2)


def kernel(x, edge_index, W, b):
    n_nodes, d = x.shape
    src = edge_index[0]
    dst = edge_index[1]
    p, dg = _sc_aggregate(x, src, dst)
    bm = 2000
    dg3 = dg.reshape(NC * NS, n_nodes // bm, bm).transpose(1, 0, 2)
    return _tc_finish(p, dg3, W, b.reshape(1, -1))


# speedup vs baseline: 11.0659x; 1.8177x over previous
import jax
import jax.numpy as jnp
from jax import lax
from jax.experimental import pallas as pl
from jax.experimental.pallas import tpu as pltpu
from jax.experimental.pallas import tpu_sc as plsc

NC = 2
NS = 16
CHUNK = 80


def _sc_body(x_hbm, src_hbm, dst_hbm, p_hbm, dg_hbm,
             acc_sh, dst_v, idx_v, rows_v, rows_w, src_a, src_b, didx_l,
             deg_l, sem, sem2):
    c = lax.axis_index("c")
    s = lax.axis_index("s")

    n_nodes = acc_sh.shape[0]
    d = rows_v.shape[1]
    total_chunks = n_nodes // CHUNK
    cpt = -(-total_chunks // NS)
    last = total_chunks - cpt * (NS - 1)
    nk = jnp.where(s == NS - 1, last, cpt)

    zero16 = jnp.zeros((16,), jnp.float32)
    zero16i = jnp.zeros((16,), jnp.int32)
    one16 = jnp.ones((16,), jnp.float32)
    iota16 = lax.iota(jnp.int32, 16)

    def _fill(i, carry):
        for j in range(d // 16):
            rows_v[i, pl.ds(j * 16, 16)] = zero16
        return carry

    lax.fori_loop(0, CHUNK, _fill, 0)

    def _zdeg(i, carry):
        deg_l[pl.ds(i * 16, 16)] = zero16
        return carry

    lax.fori_loop(0, n_nodes // 16, _zdeg, 0)

    def _set_idx(base):
        for i in range(CHUNK // 16):
            idx_v[pl.ds(i * 16, 16)] = base + (i * 16) + iota16

    def _zero_body(k, carry):
        _set_idx((s * cpt + k) * CHUNK)
        pltpu.sync_copy(rows_v, acc_sh.at[idx_v])
        return carry

    lax.fori_loop(0, nk, _zero_body, 0)

    plsc.subcore_barrier()

    n_chunks = didx_l.shape[0]
    w = c * NS + s
    ebase = w * (n_chunks * CHUNK)
    pltpu.sync_copy(dst_hbm.at[w], didx_l)

    bufs = (rows_v, rows_w)
    sbufs = (src_a, src_b)
    sems = (sem, sem2)

    def _start(ci, b):
        pltpu.sync_copy(src_hbm.at[pl.ds(ebase + ci * CHUNK, CHUNK)],
                        sbufs[b])
        pltpu.async_copy(x_hbm.at[sbufs[b]], bufs[b], sems[b])

    def _finish(ci, b):
        pltpu.make_async_copy(x_hbm.at[sbufs[b]], bufs[b],
                              sems[b]).wait()
        for j in range(CHUNK // 16):
            dst_v[pl.ds(j * 16, 16)] = didx_l[ci, pl.ds(j * 16, 16)]
        pltpu.sync_copy(bufs[b], acc_sh.at[dst_v], add=True)
        for j in range(CHUNK // 16):
            dst16 = didx_l[ci, pl.ds(j * 16, 16)]
            plsc.addupdate_scatter(deg_l, [dst16], one16)

    _start(0, 0)
    _start(1, 1)
    n_pairs = n_chunks // 2

    def pair_body(i2, carry):
        c0 = 2 * i2
        _finish(c0, 0)

        @pl.when(c0 + 2 < n_chunks)
        def _s0():
            _start(c0 + 2, 0)

        _finish(c0 + 1, 1)

        @pl.when(c0 + 3 < n_chunks)
        def _s1():
            _start(c0 + 3, 1)

        return carry

    lax.fori_loop(0, n_pairs, pair_body, 0)
    if n_chunks % 2:
        _finish(n_chunks - 1, 0)

    plsc.subcore_barrier()

    def _wb_body(k, carry):
        base = (s * cpt + k) * CHUNK
        _set_idx(base)
        pltpu.async_copy(acc_sh.at[idx_v], rows_v, sem).wait()
        hb = pl.multiple_of(base, CHUNK)
        pltpu.sync_copy(rows_v, p_hbm.at[c, pl.ds(hb, CHUNK)])
        return carry

    lax.fori_loop(0, nk, _wb_body, 0)
    pltpu.sync_copy(deg_l, dg_hbm.at[c * NS + s, 0])


def _tc_body(p_ref, dg_ref, w_ref, b_ref, o_ref):
    nw = dg_ref.shape[1]
    sdat = p_ref[0] + p_ref[1]
    ones_col = jnp.ones((nw, 1), jnp.float32)
    deg = lax.dot_general(dg_ref[0], ones_col, (((0,), (0,)), ((), ())),
                          preferred_element_type=jnp.float32)
    m = lax.dot_general(sdat, w_ref[...], (((1,), (1,)), ((), ())),
                        preferred_element_type=jnp.float32)
    o_ref[...] = (m + deg * b_ref[...]) / jnp.maximum(deg, 1.0)


def _sc_aggregate(x, src, dst):
    n_nodes, d = x.shape
    mesh = plsc.VectorSubcoreMesh(core_axis_name="c", subcore_axis_name="s",
                                  num_cores=NC, num_subcores=NS)
    f = pl.kernel(
        _sc_body,
        out_type=[
            jax.ShapeDtypeStruct((NC, n_nodes, d), jnp.float32),
            jax.ShapeDtypeStruct((NC * NS, 1, n_nodes), jnp.float32),
        ],
        mesh=mesh,
        compiler_params=pltpu.CompilerParams(needs_layout_passes=False),
        scratch_types=[
            pltpu.VMEM_SHARED((n_nodes, d), jnp.float32),
            pltpu.VMEM((CHUNK,), jnp.int32),
            pltpu.VMEM((CHUNK,), jnp.int32),
            pltpu.VMEM((CHUNK, d), jnp.float32),
            pltpu.VMEM((CHUNK, d), jnp.float32),
            pltpu.VMEM((CHUNK,), jnp.int32),
            pltpu.VMEM((CHUNK,), jnp.int32),
            pltpu.VMEM(dst.shape[1:], jnp.int32),
            pltpu.VMEM((n_nodes,), jnp.float32),
            pltpu.SemaphoreType.DMA,
            pltpu.SemaphoreType.DMA,
        ],
    )
    return f(x, src, dst)


def _tc_finish(p, dg3, w, b2):
    nc, n_nodes, d = p.shape
    nblk, nw, bm = dg3.shape
    grid = (n_nodes // bm,)
    return pl.pallas_call(
        _tc_body,
        grid=grid,
        in_specs=[
            pl.BlockSpec((nc, bm, d), lambda i: (0, i, 0)),
            pl.BlockSpec((1, nw, bm), lambda i: (i, 0, 0)),
            pl.BlockSpec((d, d), lambda i: (0, 0)),
            pl.BlockSpec((1, d), lambda i: (0, 0)),
        ],
        out_specs=pl.BlockSpec((bm, d), lambda i: (i, 0)),
        out_shape=jax.ShapeDtypeStruct((n_nodes, d), jnp.float32),
    )(p, dg3, w, b2)


def kernel(x, edge_index, W, b):
    n_nodes, d = x.shape
    src = edge_index[0]
    dst = edge_index[1]
    n_chunks = src.shape[0] // (NC * NS * CHUNK)
    dst3 = dst.reshape(NC * NS, n_chunks, CHUNK)
    p, dg = _sc_aggregate(x, src, dst3)
    bm = 2000
    dg3 = dg.reshape(NC * NS, n_nodes // bm, bm).transpose(1, 0, 2)
    return _tc_finish(p, dg3, W, b.reshape(1, -1))


# speedup vs baseline: 11.2344x; 1.0152x over previous
import jax
import jax.numpy as jnp
from jax import lax
from jax.experimental import pallas as pl
from jax.experimental.pallas import tpu as pltpu
from jax.experimental.pallas import tpu_sc as plsc

NC = 2
NS = 16
CHUNK = 80


def _sc_body(x_hbm, src_hbm, dst_hbm, p_hbm, dg_hbm,
             acc_sh, idx_v, rows_v, rows_w, src_a, src_b, didx_l,
             deg_l, sem, sem2, sem_s, sem_s2):
    c = lax.axis_index("c")
    s = lax.axis_index("s")

    n_nodes = acc_sh.shape[0]
    d = rows_v.shape[1]
    total_chunks = n_nodes // CHUNK
    cpt = -(-total_chunks // NS)
    last = total_chunks - cpt * (NS - 1)
    nk = jnp.where(s == NS - 1, last, cpt)

    zero16 = jnp.zeros((16,), jnp.float32)
    zero16i = jnp.zeros((16,), jnp.int32)
    one16 = jnp.ones((16,), jnp.float32)
    iota16 = lax.iota(jnp.int32, 16)

    def _fill(i, carry):
        for j in range(d // 16):
            rows_v[i, pl.ds(j * 16, 16)] = zero16
        return carry

    lax.fori_loop(0, CHUNK, _fill, 0)

    def _zdeg(i, carry):
        deg_l[pl.ds(i * 16, 16)] = zero16
        return carry

    lax.fori_loop(0, n_nodes // 16, _zdeg, 0)

    def _set_idx(base):
        for i in range(CHUNK // 16):
            idx_v[pl.ds(i * 16, 16)] = base + (i * 16) + iota16

    def _zero_body(k, carry):
        _set_idx((s * cpt + k) * CHUNK)
        pltpu.sync_copy(rows_v, acc_sh.at[idx_v])
        return carry

    lax.fori_loop(0, nk, _zero_body, 0)

    plsc.subcore_barrier()

    n_chunks = didx_l.shape[0]
    w = c * NS + s
    ebase = w * (n_chunks * CHUNK)
    pltpu.sync_copy(dst_hbm.at[w], didx_l)

    bufs = (rows_v, rows_w)
    sbufs = (src_a, src_b)
    gsems = (sem, sem2)
    ssems = (sem_s, sem_s2)

    def _start(ci, b):
        pltpu.sync_copy(src_hbm.at[pl.ds(ebase + ci * CHUNK, CHUNK)],
                        sbufs[b])
        pltpu.async_copy(x_hbm.at[sbufs[b]], bufs[b], gsems[b])

    def _wait_scatter(ci, b):
        pltpu.make_async_copy(bufs[b], acc_sh.at[didx_l.at[ci]],
                              ssems[b]).wait()

    def _finish(ci, b):
        pltpu.make_async_copy(x_hbm.at[sbufs[b]], bufs[b],
                              gsems[b]).wait()
        pltpu.async_copy(bufs[b], acc_sh.at[didx_l.at[ci]], ssems[b],
                         add=True)
        for j in range(CHUNK // 16):
            dst16 = didx_l[ci, pl.ds(j * 16, 16)]
            plsc.addupdate_scatter(deg_l, [dst16], one16)

    _start(0, 0)
    _start(1, 1)
    n_pairs = n_chunks // 2

    def pair_body(i2, carry):
        c0 = 2 * i2
        _finish(c0, 0)

        @pl.when(c0 + 2 < n_chunks)
        def _s0():
            _wait_scatter(c0, 0)
            _start(c0 + 2, 0)

        _finish(c0 + 1, 1)

        @pl.when(c0 + 3 < n_chunks)
        def _s1():
            _wait_scatter(c0 + 1, 1)
            _start(c0 + 3, 1)

        return carry

    lax.fori_loop(0, n_pairs, pair_body, 0)
    if n_chunks % 2:
        _finish(n_chunks - 1, 0)
        _wait_scatter(n_chunks - 1, 0)
        _wait_scatter(n_chunks - 2, 1)
    else:
        _wait_scatter(n_chunks - 2, 0)
        _wait_scatter(n_chunks - 1, 1)

    plsc.subcore_barrier()

    def _wb_body(k, carry):
        base = (s * cpt + k) * CHUNK
        _set_idx(base)
        pltpu.async_copy(acc_sh.at[idx_v], rows_v, sem).wait()
        hb = pl.multiple_of(base, CHUNK)
        pltpu.sync_copy(rows_v, p_hbm.at[c, pl.ds(hb, CHUNK)])
        return carry

    lax.fori_loop(0, nk, _wb_body, 0)
    pltpu.sync_copy(deg_l, dg_hbm.at[c * NS + s, 0])


def _tc_body(p_ref, dg_ref, w_ref, b_ref, o_ref):
    nw = dg_ref.shape[1]
    sdat = p_ref[0] + p_ref[1]
    ones_col = jnp.ones((nw, 1), jnp.float32)
    deg = lax.dot_general(dg_ref[0], ones_col, (((0,), (0,)), ((), ())),
                          preferred_element_type=jnp.float32)
    m = lax.dot_general(sdat, w_ref[...], (((1,), (1,)), ((), ())),
                        preferred_element_type=jnp.float32)
    o_ref[...] = (m + deg * b_ref[...]) / jnp.maximum(deg, 1.0)


def _sc_aggregate(x, src, dst):
    n_nodes, d = x.shape
    mesh = plsc.VectorSubcoreMesh(core_axis_name="c", subcore_axis_name="s",
                                  num_cores=NC, num_subcores=NS)
    f = pl.kernel(
        _sc_body,
        out_type=[
            jax.ShapeDtypeStruct((NC, n_nodes, d), jnp.float32),
            jax.ShapeDtypeStruct((NC * NS, 1, n_nodes), jnp.float32),
        ],
        mesh=mesh,
        compiler_params=pltpu.CompilerParams(needs_layout_passes=False),
        scratch_types=[
            pltpu.VMEM_SHARED((n_nodes, d), jnp.float32),
            pltpu.VMEM((CHUNK,), jnp.int32),
            pltpu.VMEM((CHUNK, d), jnp.float32),
            pltpu.VMEM((CHUNK, d), jnp.float32),
            pltpu.VMEM((CHUNK,), jnp.int32),
            pltpu.VMEM((CHUNK,), jnp.int32),
            pltpu.VMEM(dst.shape[1:], jnp.int32),
            pltpu.VMEM((n_nodes,), jnp.float32),
            pltpu.SemaphoreType.DMA,
            pltpu.SemaphoreType.DMA,
            pltpu.SemaphoreType.DMA,
            pltpu.SemaphoreType.DMA,
        ],
    )
    return f(x, src, dst)


def _tc_finish(p, dg3, w, b2):
    nc, n_nodes, d = p.shape
    nblk, nw, bm = dg3.shape
    grid = (n_nodes // bm,)
    return pl.pallas_call(
        _tc_body,
        grid=grid,
        in_specs=[
            pl.BlockSpec((nc, bm, d), lambda i: (0, i, 0)),
            pl.BlockSpec((1, nw, bm), lambda i: (i, 0, 0)),
            pl.BlockSpec((d, d), lambda i: (0, 0)),
            pl.BlockSpec((1, d), lambda i: (0, 0)),
        ],
        out_specs=pl.BlockSpec((bm, d), lambda i: (i, 0)),
        out_shape=jax.ShapeDtypeStruct((n_nodes, d), jnp.float32),
    )(p, dg3, w, b2)


def kernel(x, edge_index, W, b):
    n_nodes, d = x.shape
    src = edge_index[0]
    dst = edge_index[1]
    n_chunks = src.shape[0] // (NC * NS * CHUNK)
    dst3 = dst.reshape(NC * NS, n_chunks, CHUNK)
    p, dg = _sc_aggregate(x, src, dst3)
    bm = 2000
    dg3 = dg.reshape(NC * NS, n_nodes // bm, bm).transpose(1, 0, 2)
    return _tc_finish(p, dg3, W, b.reshape(1, -1))


# speedup vs baseline: 11.6990x; 1.0414x over previous
import jax
import jax.numpy as jnp
from jax import lax
from jax.experimental import pallas as pl
from jax.experimental.pallas import tpu as pltpu
from jax.experimental.pallas import tpu_sc as plsc

NC = 2
NS = 16
CHUNK = 80


def _sc_body(x_hbm, src_hbm, dst_hbm, p_hbm, dg_hbm,
             acc_sh, idx2d, rows_v, rows_w, src_a, src_b, didx_l,
             deg_l, sem, sem2, sem_s, sem_s2):
    c = lax.axis_index("c")
    s = lax.axis_index("s")

    n_nodes = acc_sh.shape[0]
    d = rows_v.shape[1]
    total_chunks = n_nodes // CHUNK
    cpt = -(-total_chunks // NS)

    zero16 = jnp.zeros((16,), jnp.float32)
    one16 = jnp.ones((16,), jnp.float32)
    iota16 = lax.iota(jnp.int32, 16)

    def _guard(k, fn):
        if (k + 1) * NS <= total_chunks:
            fn()
        else:
            pl.when(k * NS + s < total_chunks)(fn)

    def _fill(i, carry):
        for j in range(d // 16):
            rows_v[i, pl.ds(j * 16, 16)] = zero16
        return carry

    lax.fori_loop(0, CHUNK, _fill, 0)

    for k in range(cpt):
        base = (k * NS + s) * CHUNK
        for i in range(CHUNK // 16):
            idx2d[k, pl.ds(i * 16, 16)] = base + (i * 16) + iota16

    for k in range(cpt):
        def _zgo(k=k):
            pltpu.async_copy(rows_v, acc_sh.at[idx2d.at[k]], sem_s)
        _guard(k, _zgo)

    def _zdeg(i, carry):
        deg_l[pl.ds(i * 16, 16)] = zero16
        return carry

    lax.fori_loop(0, n_nodes // 16, _zdeg, 0)

    for k in range(cpt):
        def _zwait(k=k):
            pltpu.make_async_copy(rows_v, acc_sh.at[idx2d.at[k]],
                                  sem_s).wait()
        _guard(k, _zwait)

    plsc.subcore_barrier()

    n_chunks = didx_l.shape[0]
    w = c * NS + s
    ebase = w * (n_chunks * CHUNK)
    pltpu.sync_copy(dst_hbm.at[w], didx_l)

    bufs = (rows_v, rows_w)
    sbufs = (src_a, src_b)
    gsems = (sem, sem2)
    ssems = (sem_s, sem_s2)

    def _start(ci, b):
        pltpu.sync_copy(src_hbm.at[pl.ds(ebase + ci * CHUNK, CHUNK)],
                        sbufs[b])
        pltpu.async_copy(x_hbm.at[sbufs[b]], bufs[b], gsems[b])

    def _wait_scatter(ci, b):
        pltpu.make_async_copy(bufs[b], acc_sh.at[didx_l.at[ci]],
                              ssems[b]).wait()

    def _finish(ci, b):
        pltpu.make_async_copy(x_hbm.at[sbufs[b]], bufs[b],
                              gsems[b]).wait()
        pltpu.async_copy(bufs[b], acc_sh.at[didx_l.at[ci]], ssems[b],
                         add=True)
        for j in range(CHUNK // 16):
            dst16 = didx_l[ci, pl.ds(j * 16, 16)]
            plsc.addupdate_scatter(deg_l, [dst16], one16)

    _start(0, 0)
    _start(1, 1)
    n_pairs = n_chunks // 2

    def pair_body(i2, carry):
        c0 = 2 * i2
        _finish(c0, 0)

        @pl.when(c0 + 2 < n_chunks)
        def _s0():
            _wait_scatter(c0, 0)
            _start(c0 + 2, 0)

        _finish(c0 + 1, 1)

        @pl.when(c0 + 3 < n_chunks)
        def _s1():
            _wait_scatter(c0 + 1, 1)
            _start(c0 + 3, 1)

        return carry

    lax.fori_loop(0, n_pairs, pair_body, 0)
    if n_chunks % 2:
        _finish(n_chunks - 1, 0)
        _wait_scatter(n_chunks - 1, 0)
        _wait_scatter(n_chunks - 2, 1)
    else:
        _wait_scatter(n_chunks - 2, 0)
        _wait_scatter(n_chunks - 1, 1)

    plsc.subcore_barrier()

    def _hslice(k):
        hb = pl.multiple_of((k * NS + s) * CHUNK, 8)
        return p_hbm.at[c, pl.ds(hb, CHUNK)]

    def _wb_gather(k, b):
        pltpu.async_copy(acc_sh.at[idx2d.at[k]], bufs[b], gsems[b])

    def _wb_gwait(k, b):
        pltpu.make_async_copy(acc_sh.at[idx2d.at[k]], bufs[b],
                              gsems[b]).wait()

    def _wb_write(k, b):
        pltpu.async_copy(bufs[b], _hslice(k), ssems[b])

    def _wb_wwait(k, b):
        pltpu.make_async_copy(bufs[b], _hslice(k), ssems[b]).wait()

    def _wb_g0():
        _wb_gather(0, 0)

    def _wb_g1():
        _wb_gather(1, 1)

    _guard(0, _wb_g0)
    _guard(1, _wb_g1)
    for k in range(cpt):
        b = k % 2

        def _step(k=k, b=b):
            _wb_gwait(k, b)
            _wb_write(k, b)

        _guard(k, _step)
        if k + 2 < cpt:
            def _next(k=k, b=b):
                _wb_wwait(k, b)
                _wb_gather(k + 2, b)

            _guard(k + 2, _next)
    for k in range(cpt):
        if k + 2 >= cpt:
            _guard(k, lambda k=k: _wb_wwait(k, k % 2))
        elif (k + 3) * NS > total_chunks:
            def _residual(k=k):
                _wb_wwait(k, k % 2)

            pl.when(jnp.logical_and(k * NS + s < total_chunks,
                                    (k + 2) * NS + s >= total_chunks))(
                _residual)
    pltpu.sync_copy(deg_l, dg_hbm.at[c * NS + s, 0])


def _tc_body(p_ref, dg_ref, w_ref, b_ref, o_ref):
    nw = dg_ref.shape[1]
    sdat = p_ref[0] + p_ref[1]
    ones_col = jnp.ones((nw, 1), jnp.float32)
    deg = lax.dot_general(dg_ref[0], ones_col, (((0,), (0,)), ((), ())),
                          preferred_element_type=jnp.float32)
    m = lax.dot_general(sdat, w_ref[...], (((1,), (1,)), ((), ())),
                        preferred_element_type=jnp.float32)
    o_ref[...] = (m + deg * b_ref[...]) / jnp.maximum(deg, 1.0)


def _sc_aggregate(x, src, dst):
    n_nodes, d = x.shape
    mesh = plsc.VectorSubcoreMesh(core_axis_name="c", subcore_axis_name="s",
                                  num_cores=NC, num_subcores=NS)
    f = pl.kernel(
        _sc_body,
        out_type=[
            jax.ShapeDtypeStruct((NC, n_nodes, d), jnp.float32),
            jax.ShapeDtypeStruct((NC * NS, 1, n_nodes), jnp.float32),
        ],
        mesh=mesh,
        compiler_params=pltpu.CompilerParams(needs_layout_passes=False),
        scratch_types=[
            pltpu.VMEM_SHARED((n_nodes, d), jnp.float32),
            pltpu.VMEM((-(-(n_nodes // CHUNK) // NS), CHUNK), jnp.int32),
            pltpu.VMEM((CHUNK, d), jnp.float32),
            pltpu.VMEM((CHUNK, d), jnp.float32),
            pltpu.VMEM((CHUNK,), jnp.int32),
            pltpu.VMEM((CHUNK,), jnp.int32),
            pltpu.VMEM(dst.shape[1:], jnp.int32),
            pltpu.VMEM((n_nodes,), jnp.float32),
            pltpu.SemaphoreType.DMA,
            pltpu.SemaphoreType.DMA,
            pltpu.SemaphoreType.DMA,
            pltpu.SemaphoreType.DMA,
        ],
    )
    return f(x, src, dst)


def _tc_finish(p, dg3, w, b2):
    nc, n_nodes, d = p.shape
    nblk, nw, bm = dg3.shape
    grid = (n_nodes // bm,)
    return pl.pallas_call(
        _tc_body,
        grid=grid,
        in_specs=[
            pl.BlockSpec((nc, bm, d), lambda i: (0, i, 0)),
            pl.BlockSpec((1, nw, bm), lambda i: (i, 0, 0)),
            pl.BlockSpec((d, d), lambda i: (0, 0)),
            pl.BlockSpec((1, d), lambda i: (0, 0)),
        ],
        out_specs=pl.BlockSpec((bm, d), lambda i: (i, 0)),
        out_shape=jax.ShapeDtypeStruct((n_nodes, d), jnp.float32),
    )(p, dg3, w, b2)


def kernel(x, edge_index, W, b):
    n_nodes, d = x.shape
    src = edge_index[0]
    dst = edge_index[1]
    n_chunks = src.shape[0] // (NC * NS * CHUNK)
    dst3 = dst.reshape(NC * NS, n_chunks, CHUNK)
    p, dg = _sc_aggregate(x, src, dst3)
    bm = 2000
    dg3 = dg.reshape(NC * NS, n_nodes // bm, bm).transpose(1, 0, 2)
    return _tc_finish(p, dg3, W, b.reshape(1, -1))
